# baseline (device time: 81074 ns/iter reference)
import jax
import jax.numpy as jnp
from jax import lax
from jax.experimental import pallas as pl
from jax.experimental.pallas import tpu as pltpu

N_DEV = 4


def _gelu(y):
    c = 0.7978845608028654
    return 0.5 * y * (1.0 + jnp.tanh(c * (y + 0.044715 * y * y * y)))


def kernel(x, w_mat):
    m_per, k = x.shape
    _, n_per = w_mat.shape
    H = m_per // 2
    T = m_per // 8
    E = 16

    def body(x_ref, w_ref, out_ref, gx_ref, send_sems, recv_sems):
        my = lax.axis_index("i")
        left = (my - 1) % N_DEV
        right = (my + 1) % N_DEV
        across = (my + 2) % N_DEV

        def copy(src, dst, slot, dev):
            return pltpu.make_async_remote_copy(
                src_ref=src, dst_ref=dst,
                send_sem=send_sems.at[slot], recv_sem=recv_sems.at[slot],
                device_id=(dev,), device_id_type=pl.DeviceIdType.MESH,
            )

        def recv(dst, slot):
            return copy(dst, dst, slot, left)

        def mm(origin, r0, nrows):
            y = jnp.dot(
                gx_ref[origin, pl.ds(r0, nrows)], w_ref[:, :],
                preferred_element_type=jnp.float32,
            )
            out_ref[pl.ds(origin * m_per + r0, nrows), :] = _gelu(y)

        barrier_sem = pltpu.get_barrier_semaphore()
        for nbr in [left, right]:
            pl.semaphore_signal(
                barrier_sem, inc=1,
                device_id=(nbr,), device_id_type=pl.DeviceIdType.MESH,
            )
        pl.semaphore_wait(barrier_sem, 2)

        sR1 = copy(x_ref.at[pl.ds(0, H + E)],
                   gx_ref.at[my, pl.ds(0, H + E)], 0, right)
        sR1.start()
        sL1 = copy(x_ref.at[pl.ds(H - E, H + E)],
                   gx_ref.at[my, pl.ds(H - E, H + E)], 4, left)
        sL1.start()

        y = jnp.dot(x_ref[:, :], w_ref[:, :], preferred_element_type=jnp.float32)
        out_ref[pl.ds(my * m_per, m_per), :] = _gelu(y)

        recv(gx_ref.at[left, pl.ds(0, H + E)], 0).wait_recv()
        sRF = copy(gx_ref.at[left, pl.ds(0, H)],
                   gx_ref.at[left, pl.ds(0, H)], 1, right)
        sRF.start()
        sR2 = copy(x_ref.at[pl.ds(H + E, m_per - T - H - E)],
                   gx_ref.at[my, pl.ds(H + E, m_per - T - H - E)], 2, right)
        sR2.start()
        sR3 = copy(x_ref.at[pl.ds(m_per - T, T)],
                   gx_ref.at[my, pl.ds(m_per - T, T)], 3, right)
        sR3.start()

        recv(gx_ref.at[right, pl.ds(H - E, H + E)], 4).wait_recv()
        sLF = copy(gx_ref.at[right, pl.ds(H, H)],
                   gx_ref.at[right, pl.ds(H, H)], 5, left)
        sLF.start()
        sL2 = copy(x_ref.at[pl.ds(T, H - E - T)],
                   gx_ref.at[my, pl.ds(T, H - E - T)], 6, left)
        sL2.start()
        sL3 = copy(x_ref.at[pl.ds(0, T)], gx_ref.at[my, pl.ds(0, T)], 7, left)
        sL3.start()

        mm(left, 0, H + E)
        mm(right, H - E, H + E)

        recv(gx_ref.at[across, pl.ds(0, H)], 1).wait_recv()
        recv(gx_ref.at[across, pl.ds(H, H)], 5).wait_recv()
        mm(across, 0, m_per)

        recv(gx_ref.at[left, pl.ds(H + E, m_per - T - H - E)], 2).wait_recv()
        mm(left, H + E, m_per - T - H - E)
        recv(gx_ref.at[right, pl.ds(T, H - E - T)], 6).wait_recv()
        mm(right, T, H - E - T)

        recv(gx_ref.at[left, pl.ds(m_per - T, T)], 3).wait_recv()
        mm(left, m_per - T, T)
        recv(gx_ref.at[right, pl.ds(0, T)], 7).wait_recv()
        mm(right, 0, T)

        for s in [sR1, sRF, sR2, sR3, sL1, sLF, sL2, sL3]:
            s.wait_send()

    return pl.pallas_call(
        body,
        out_shape=jax.ShapeDtypeStruct((N_DEV * m_per, n_per), jnp.float32),
        in_specs=[
            pl.BlockSpec(memory_space=pltpu.VMEM),
            pl.BlockSpec(memory_space=pltpu.VMEM),
        ],
        out_specs=pl.BlockSpec(memory_space=pltpu.VMEM),
        scratch_shapes=[
            pltpu.VMEM((N_DEV, m_per, k), x.dtype),
            pltpu.SemaphoreType.DMA((8,)),
            pltpu.SemaphoreType.DMA((8,)),
        ],
        compiler_params=pltpu.CompilerParams(collective_id=0),
    )(x, w_mat)


# device time: 80957 ns/iter; 1.0014x vs baseline; 1.0014x over previous
import jax
import jax.numpy as jnp
from jax import lax
from jax.experimental import pallas as pl
from jax.experimental.pallas import tpu as pltpu

N_DEV = 4


def _gelu(y):
    c = 0.7978845608028654
    return 0.5 * y * (1.0 + jnp.tanh(c * (y + 0.044715 * y * y * y)))


def kernel(x, w_mat):
    m_per, k = x.shape
    _, n_per = w_mat.shape
    H = m_per // 2
    T = m_per // 16
    E = 16

    def body(x_ref, w_ref, out_ref, gx_ref, send_sems, recv_sems):
        my = lax.axis_index("i")
        left = (my - 1) % N_DEV
        right = (my + 1) % N_DEV
        across = (my + 2) % N_DEV

        def copy(src, dst, slot, dev):
            return pltpu.make_async_remote_copy(
                src_ref=src, dst_ref=dst,
                send_sem=send_sems.at[slot], recv_sem=recv_sems.at[slot],
                device_id=(dev,), device_id_type=pl.DeviceIdType.MESH,
            )

        def recv(dst, slot):
            return copy(dst, dst, slot, left)

        def mm(origin, r0, nrows):
            y = jnp.dot(
                gx_ref[origin, pl.ds(r0, nrows)], w_ref[:, :],
                preferred_element_type=jnp.float32,
            )
            out_ref[pl.ds(origin * m_per + r0, nrows), :] = _gelu(y)

        barrier_sem = pltpu.get_barrier_semaphore()
        for nbr in [left, right]:
            pl.semaphore_signal(
                barrier_sem, inc=1,
                device_id=(nbr,), device_id_type=pl.DeviceIdType.MESH,
            )
        pl.semaphore_wait(barrier_sem, 2)

        sR1 = copy(x_ref.at[pl.ds(0, H + E)],
                   gx_ref.at[my, pl.ds(0, H + E)], 0, right)
        sR1.start()
        sL1 = copy(x_ref.at[pl.ds(H - E, H + E)],
                   gx_ref.at[my, pl.ds(H - E, H + E)], 4, left)
        sL1.start()

        y = jnp.dot(x_ref[:, :], w_ref[:, :], preferred_element_type=jnp.float32)
        out_ref[pl.ds(my * m_per, m_per), :] = _gelu(y)

        recv(gx_ref.at[left, pl.ds(0, H + E)], 0).wait_recv()
        sRF = copy(gx_ref.at[left, pl.ds(0, H)],
                   gx_ref.at[left, pl.ds(0, H)], 1, right)
        sRF.start()
        sR2 = copy(x_ref.at[pl.ds(H + E, m_per - T - H - E)],
                   gx_ref.at[my, pl.ds(H + E, m_per - T - H - E)], 2, right)
        sR2.start()
        sR3 = copy(x_ref.at[pl.ds(m_per - T, T)],
                   gx_ref.at[my, pl.ds(m_per - T, T)], 3, right)
        sR3.start()

        recv(gx_ref.at[right, pl.ds(H - E, H + E)], 4).wait_recv()
        sLF = copy(gx_ref.at[right, pl.ds(H, H)],
                   gx_ref.at[right, pl.ds(H, H)], 5, left)
        sLF.start()
        sL2 = copy(x_ref.at[pl.ds(T, H - E - T)],
                   gx_ref.at[my, pl.ds(T, H - E - T)], 6, left)
        sL2.start()
        sL3 = copy(x_ref.at[pl.ds(0, T)], gx_ref.at[my, pl.ds(0, T)], 7, left)
        sL3.start()

        mm(left, 0, H + E)
        mm(right, H - E, H + E)

        recv(gx_ref.at[across, pl.ds(0, H)], 1).wait_recv()
        recv(gx_ref.at[across, pl.ds(H, H)], 5).wait_recv()
        mm(across, 0, m_per)

        recv(gx_ref.at[left, pl.ds(H + E, m_per - T - H - E)], 2).wait_recv()
        mm(left, H + E, m_per - T - H - E)
        recv(gx_ref.at[right, pl.ds(T, H - E - T)], 6).wait_recv()
        mm(right, T, H - E - T)

        recv(gx_ref.at[left, pl.ds(m_per - T, T)], 3).wait_recv()
        mm(left, m_per - T, T)
        recv(gx_ref.at[right, pl.ds(0, T)], 7).wait_recv()
        mm(right, 0, T)

        for s in [sR1, sRF, sR2, sR3, sL1, sLF, sL2, sL3]:
            s.wait_send()

    return pl.pallas_call(
        body,
        out_shape=jax.ShapeDtypeStruct((N_DEV * m_per, n_per), jnp.float32),
        in_specs=[
            pl.BlockSpec(memory_space=pltpu.VMEM),
            pl.BlockSpec(memory_space=pltpu.VMEM),
        ],
        out_specs=pl.BlockSpec(memory_space=pltpu.VMEM),
        scratch_shapes=[
            pltpu.VMEM((N_DEV, m_per, k), x.dtype),
            pltpu.SemaphoreType.DMA((8,)),
            pltpu.SemaphoreType.DMA((8,)),
        ],
        compiler_params=pltpu.CompilerParams(collective_id=0),
    )(x, w_mat)
